# baseline (speedup 1.0000x reference)
import functools

import jax
import jax.numpy as jnp
from jax import lax
from jax.experimental import pallas as pl
from jax.experimental.pallas import tpu as pltpu
from jax.experimental.pallas import tpu_sc as plsc

N_USERS = 2000
N_ITEMS = 4000
N_NODES = 10000
EMB = 128
RELD = 64
E = 320000
N_LAYERS = 2

NC = 2
NS = 16
NW = NC * NS
K = 48
NBUF = 6
NCHUNK = 210
EPW = K * NCHUNK
E_PAD = NW * EPW
LOOK = 4
SLAB = 624
LAST = N_NODES - 15 * SLAB
NVEC = EMB // 16

_MESH = plsc.VectorSubcoreMesh(
    core_axis_name="c", subcore_axis_name="s", num_cores=NC, num_subcores=NS)


@functools.partial(
    pl.kernel,
    out_type=jax.ShapeDtypeStruct((NC, N_NODES, EMB), jnp.float32),
    mesh=_MESH,
    scratch_types=[
        pltpu.VMEM((NBUF, K), jnp.int32),
        pltpu.VMEM((NBUF, K), jnp.int32),
        pltpu.VMEM((NBUF, K), jnp.float32),
        pltpu.VMEM((NBUF, K, EMB), jnp.float32),
        pltpu.VMEM_SHARED((N_NODES, EMB), jnp.float32),
        pltpu.SemaphoreType.DMA((NBUF,)),
        pltpu.SemaphoreType.DMA((NBUF,)),
    ],
)
def _spmm(ego_hbm, col_hbm, row_hbm, val_hbm, zero_hbm, out_hbm,
          colr, rowr, valr, rowsr, acc, gsem, ssem):
    c = lax.axis_index("c")
    s = lax.axis_index("s")
    wid = s * NC + c
    ebase = wid * EPW

    @pl.when(s < 15)
    def _():
        pltpu.sync_copy(zero_hbm.at[pl.ds(0, SLAB)],
                        acc.at[pl.ds(s * SLAB, SLAB)])

    @pl.when(s == 15)
    def _():
        pltpu.sync_copy(zero_hbm, acc.at[pl.ds(15 * SLAB, LAST)])

    plsc.subcore_barrier()

    def fetch_issue(i, b):
        base = ebase + i * K
        pltpu.sync_copy(col_hbm.at[pl.ds(base, K)], colr.at[b])
        pltpu.async_copy(row_hbm.at[pl.ds(base, K)], rowr.at[b], gsem.at[b])
        pltpu.async_copy(val_hbm.at[pl.ds(base, K)], valr.at[b], gsem.at[b])
        pltpu.async_copy(ego_hbm.at[colr.at[b]], rowsr.at[b], gsem.at[b])

    def fetch_wait(b):
        pltpu.make_async_copy(row_hbm.at[pl.ds(0, K)], rowr.at[b],
                              gsem.at[b]).wait()
        pltpu.make_async_copy(val_hbm.at[pl.ds(0, K)], valr.at[b],
                              gsem.at[b]).wait()
        pltpu.make_async_copy(ego_hbm.at[pl.ds(0, K)], rowsr.at[b],
                              gsem.at[b]).wait()

    def scat_issue(b):
        pltpu.async_copy(rowsr.at[b], acc.at[rowr.at[b]], ssem.at[b],
                         add=True)

    def scat_wait(b):
        pltpu.make_async_copy(rowsr.at[b], acc.at[pl.ds(0, K)],
                              ssem.at[b]).wait()

    def scale(b):
        buf = rowsr.at[b]
        valb = valr.at[b]

        def s16(jj, c2):
            off = pl.multiple_of(jj * 16, 16)
            vals16 = valb[pl.ds(off, 16)]
            for l in range(16):
                j = off + l
                v = vals16[l]
                for g in range(NVEC):
                    sl = pl.ds(g * 16, 16)
                    buf[j, sl] = buf[j, sl] * v
            return c2

        lax.fori_loop(0, K // 16, s16, 0)

    for i in range(LOOK):
        fetch_issue(i, i)

    def proc(cix, b):
        fetch_wait(b)
        scale(b)
        scat_issue(b)

        @pl.when(cix <= NCHUNK - 1 - LOOK)
        def _():
            b2 = (b + LOOK) % NBUF

            @pl.when(cix >= NBUF - LOOK)
            def _():
                scat_wait(b2)

            fetch_issue(cix + LOOK, b2)

    def body(t, carry):
        for b in range(NBUF):
            proc(t * NBUF + b, b)
        return carry

    lax.fori_loop(0, NCHUNK // NBUF, body, 0)
    for b in range(NBUF):
        scat_wait(b)
    plsc.subcore_barrier()

    @pl.when(s < 15)
    def _():
        pltpu.sync_copy(acc.at[pl.ds(s * SLAB, SLAB)],
                        out_hbm.at[c, pl.ds(s * SLAB, SLAB)])

    @pl.when(s == 15)
    def _():
        pltpu.sync_copy(acc.at[pl.ds(15 * SLAB, LAST)],
                        out_hbm.at[c, pl.ds(15 * SLAB, LAST)])


def _dense_body(ego_ref, np_ref, wr_ref, re_ref, w1t_ref, w3t_ref, out_ref):
    ego = ego_ref[...]
    neigh = np_ref[0] + np_ref[1]
    wr = wr_ref[...]
    h = jnp.dot(ego, wr, preferred_element_type=jnp.float32)
    t = jnp.dot(neigh, wr, preferred_element_type=jnp.float32)
    score = jnp.sum(t * jnp.tanh(h + re_ref[...]), axis=1, keepdims=True)
    m = jnp.max(score)
    ex = jnp.exp(score - m)
    neigh = neigh * (ex / jnp.sum(ex))
    a = jnp.dot(ego + neigh, w1t_ref[...], preferred_element_type=jnp.float32)
    b = jnp.dot(ego * neigh, w3t_ref[...], preferred_element_type=jnp.float32)
    out_ref[...] = (jnp.where(a >= 0, a, 0.2 * a)
                    + jnp.where(b >= 0, b, 0.2 * b))


def _dense(ego, neigh_parts, wr, re_, w1t, w3t):
    return pl.pallas_call(
        _dense_body,
        out_shape=jax.ShapeDtypeStruct((N_NODES, EMB), jnp.float32),
    )(ego, neigh_parts, wr, re_, w1t, w3t)


def kernel(ent_emb, rel_emb, rel_proj, W1, W3, edge_val, edge_row, edge_col):
    wr = rel_proj[0].reshape(EMB, RELD)
    re_ = rel_emb[0].reshape(1, RELD)
    w1t = W1.T
    w3t = W3.T
    zeros = jnp.zeros((LAST, EMB), jnp.float32)
    pad = E_PAD - E
    colp = jnp.concatenate([edge_col, jnp.zeros((pad,), jnp.int32)])
    rowp = jnp.concatenate([edge_row, jnp.zeros((pad,), jnp.int32)])
    valp = jnp.concatenate([edge_val, jnp.zeros((pad,), jnp.float32)])
    ego = ent_emb
    outs = [ent_emb]
    for _ in range(N_LAYERS):
        parts = _spmm(ego, colp, rowp, valp, zeros)
        ego = _dense(ego, parts, wr, re_, w1t, w3t)
        outs.append(ego)
    fin = jnp.concatenate(outs, axis=1)
    return fin[:N_USERS], fin[N_USERS:N_USERS + N_ITEMS]
